# baseline (speedup 1.0000x reference)
import functools

import jax
import jax.numpy as jnp
from jax import lax
from jax.experimental import pallas as pl
from jax.experimental.pallas import tpu as pltpu
from jax.experimental.pallas import tpu_sc as plsc

NG = 1112
NPG = 9
NV = NG * NPG
NE = NV * 16
D = 256
H = 128
NC = 2
NS = 16
NP = 10240
TRASH = NV
CHUNK = 128
NCH = 40
EPAD = NC * NS * NCH * CHUNK - NE
STRIPE = NP // NS


def _mesh():
    return plsc.VectorSubcoreMesh(
        core_axis_name="c", subcore_axis_name="s", num_cores=NC, num_subcores=NS
    )


GE = NCH * CHUNK


def _deg_body(ei_hbm, cnt_hbm, idx_v, hist_v, red_v, out_v, red_sh):
    c = lax.axis_index("c")
    s = lax.axis_index("s")

    def zero(i, carry):
        hist_v[pl.ds(i * 16, 16)] = jnp.zeros((16,), jnp.float32)
        return carry

    lax.fori_loop(0, NP // 16, zero, 0)
    ones = jnp.ones((16,), jnp.float32)
    for gi in range(2):
        g = s + NS * gi
        pltpu.sync_copy(ei_hbm.at[c, g], idx_v)

        def chunk(j, carry):
            idx = idx_v[pl.ds(j * 16, 16)]
            plsc.addupdate_scatter(hist_v, [idx], ones)
            return carry

        lax.fori_loop(0, GE // 16, chunk, 0)
    pltpu.sync_copy(hist_v, red_sh.at[s])
    plsc.subcore_barrier()
    pltpu.sync_copy(red_sh.at[:, pl.ds(s * STRIPE, STRIPE)], red_v)

    def red(k, carry):
        acc = jnp.zeros((16,), jnp.float32)
        for r in range(NS):
            acc = acc + red_v[r, pl.ds(k * 16, 16)]
        out_v[pl.ds(k * 16, 16)] = acc
        return carry

    lax.fori_loop(0, STRIPE // 16, red, 0)
    pltpu.sync_copy(out_v, cnt_hbm.at[c, pl.ds(s * STRIPE, STRIPE)])


def _degrees(ei_flat):
    return pl.kernel(
        _deg_body,
        out_type=jax.ShapeDtypeStruct((NC, NP), jnp.float32),
        mesh=_mesh(),
        scratch_types=[
            pltpu.VMEM((GE,), jnp.int32),
            pltpu.VMEM((NP,), jnp.float32),
            pltpu.VMEM((NS, STRIPE), jnp.float32),
            pltpu.VMEM((STRIPE,), jnp.float32),
            pltpu.VMEM_SHARED((NS, NP), jnp.float32),
        ],
        compiler_params=pltpu.CompilerParams(needs_layout_passes=False),
    )(ei_flat)


NT = 2 * NCH


SCH = 64
NB = 4
NCHS = GE // SCH


def _spmm_body4(h_hbm, ei_hbm, z_hbm, agg_hbm, sidx, didx,
                rows_0, rows_1, rows_2, rows_3, agg_sh,
                gs0, gs1, gs2, gs3, ss0, ss1, ss2, ss3):
    c = lax.axis_index("c")
    s = lax.axis_index("s")
    rows = (rows_0, rows_1, rows_2, rows_3)
    gsems = (gs0, gs1, gs2, gs3)
    ssems = (ss0, ss1, ss2, ss3)
    pltpu.sync_copy(z_hbm, agg_sh.at[pl.ds(s * STRIPE, STRIPE)])
    plsc.subcore_barrier()

    def gather(j, k):
        pltpu.async_copy(h_hbm.at[c].at[sidx.at[pl.ds(j * SCH, SCH)]],
                         rows[k], gsems[k])

    def gwait(j, k):
        pltpu.make_async_copy(h_hbm.at[c].at[sidx.at[pl.ds(j * SCH, SCH)]],
                              rows[k], gsems[k]).wait()

    def sstart(j, k):
        pltpu.async_copy(rows[k], agg_sh.at[didx.at[pl.ds(j * SCH, SCH)]],
                         ssems[k], add=True)

    def swait(j, k):
        pltpu.make_async_copy(rows[k], agg_sh.at[didx.at[pl.ds(j * SCH, SCH)]],
                              ssems[k]).wait()

    for gi in range(2):
        g = s + NS * gi
        pltpu.sync_copy(ei_hbm.at[0, g], sidx)
        pltpu.sync_copy(ei_hbm.at[1, g], didx)
        gather(0, 0)

        def body(t, carry):
            for k in range(NB):
                j = NB * t + k
                gwait(j, k)
                sstart(j, k)
                k1 = (k + 1) % NB

                @pl.when(j + 1 < NCHS)
                def _():
                    @pl.when(j >= 3)
                    def _():
                        swait(j - 3, k1)

                    gather(j + 1, k1)

            return carry

        lax.fori_loop(0, NCHS // NB, body, 0)
        for k in range(NB):
            swait(NCHS - NB + k, k)
    plsc.subcore_barrier()
    pltpu.sync_copy(
        agg_sh.at[pl.ds(s * STRIPE, STRIPE)],
        agg_hbm.at[c, pl.ds(s * STRIPE, STRIPE)],
    )


def _spmm(h, ei):
    return pl.kernel(
        _spmm_body4,
        out_type=jax.ShapeDtypeStruct((NC, NP, H), jnp.float32),
        mesh=_mesh(),
        scratch_types=[
            pltpu.VMEM((GE,), jnp.int32),
            pltpu.VMEM((GE,), jnp.int32),
            pltpu.VMEM((SCH, H), jnp.float32),
            pltpu.VMEM((SCH, H), jnp.float32),
            pltpu.VMEM((SCH, H), jnp.float32),
            pltpu.VMEM((SCH, H), jnp.float32),
            pltpu.VMEM_SHARED((NP, H), jnp.float32),
            pltpu.SemaphoreType.DMA,
            pltpu.SemaphoreType.DMA,
            pltpu.SemaphoreType.DMA,
            pltpu.SemaphoreType.DMA,
            pltpu.SemaphoreType.DMA,
            pltpu.SemaphoreType.DMA,
            pltpu.SemaphoreType.DMA,
            pltpu.SemaphoreType.DMA,
        ],
    )(h, ei, jnp.zeros((STRIPE, H), jnp.float32))


NGP = 1280
CTRASH = NGP - 1
GSTRIPE = NGP // NS


def _spmm_first_body(h_hbm, ei_hbm, tab_hbm, z_hbm, agg_hbm, sidx_f, didx_f,
                     csrc, cdst, tab_v, rows_a, rows_b, agg_sh,
                     gsem_a, gsem_b, ssem_a, ssem_b):
    c = lax.axis_index("c")
    s = lax.axis_index("s")
    pltpu.sync_copy(z_hbm.at[pl.ds(0, GSTRIPE)],
                    agg_sh.at[pl.ds(s * GSTRIPE, GSTRIPE)])
    pltpu.sync_copy(tab_hbm, tab_v)

    def fill(i, carry):
        csrc[pl.ds(i * 16, 16)] = jnp.zeros((16,), jnp.int32)
        cdst[pl.ds(i * 16, 16)] = jnp.full((16,), CTRASH, jnp.int32)
        return carry

    lax.fori_loop(0, (2 * GE) // 16, fill, 0)

    def compact(i, o):
        s16 = sidx_f[pl.ds(i * 16, 16)]
        d16 = didx_f[pl.ds(i * 16, 16)]
        dv = plsc.load_gather(tab_v, [d16])
        mask = dv != CTRASH
        plsc.store_compressed(csrc.at[pl.ds(o, 16)], s16, mask=mask)
        plsc.store_compressed(cdst.at[pl.ds(o, 16)], dv, mask=mask)
        return o + jnp.max(plsc.all_reduce_population_count(mask))

    o = jnp.int32(0)
    for gi in range(2):
        g = s + NS * gi
        pltpu.sync_copy(ei_hbm.at[0, g], sidx_f)
        pltpu.sync_copy(ei_hbm.at[1, g], didx_f)
        o = lax.fori_loop(0, GE // 16, compact, o)
    nch = (o + CHUNK - 1) // CHUNK
    plsc.subcore_barrier()

    def chunk(j, carry):
        pltpu.sync_copy(h_hbm.at[c].at[csrc.at[pl.ds(j * CHUNK, CHUNK)]],
                        rows_a)
        pltpu.sync_copy(rows_a, agg_sh.at[cdst.at[pl.ds(j * CHUNK, CHUNK)]],
                        add=True)
        return carry

    lax.fori_loop(0, nch, chunk, 0)
    plsc.subcore_barrier()
    pltpu.sync_copy(
        agg_sh.at[pl.ds(s * GSTRIPE, GSTRIPE)],
        agg_hbm.at[c, pl.ds(s * GSTRIPE, GSTRIPE)],
    )


def _spmm_first(h, ei_flat, tab):
    return pl.kernel(
        _spmm_first_body,
        out_type=jax.ShapeDtypeStruct((NC, NGP, H), jnp.float32),
        mesh=_mesh(),
        scratch_types=[
            pltpu.VMEM((GE,), jnp.int32),
            pltpu.VMEM((GE,), jnp.int32),
            pltpu.VMEM((2 * GE,), jnp.int32),
            pltpu.VMEM((2 * GE,), jnp.int32),
            pltpu.VMEM((NP,), jnp.int32),
            pltpu.VMEM((CHUNK, H), jnp.float32),
            pltpu.VMEM((CHUNK, H), jnp.float32),
            pltpu.VMEM_SHARED((NGP, H), jnp.float32),
            pltpu.SemaphoreType.DMA,
            pltpu.SemaphoreType.DMA,
            pltpu.SemaphoreType.DMA,
            pltpu.SemaphoreType.DMA,
        ],
        compiler_params=pltpu.CompilerParams(needs_layout_passes=False),
    )(h, ei_flat, tab, jnp.zeros((GSTRIPE, H), jnp.float32))


RB = 512


def _prep_body(x_ref, co_ref, ci_ref, h_ref, sin_ref, sout_ref):
    so = lax.rsqrt(jnp.maximum(co_ref[...], 1.0))
    si = lax.rsqrt(jnp.maximum(ci_ref[...], 1.0))
    sin_ref[...] = si
    sout_ref[...] = so
    xs = x_ref[...] * so
    h_ref[0] = xs[:, :H]
    h_ref[1] = xs[:, H:]


def _prep(x, cnt_out, cnt_in):
    return pl.pallas_call(
        _prep_body,
        grid=(NP // RB,),
        in_specs=[
            pl.BlockSpec((RB, D), lambda i: (i, 0)),
            pl.BlockSpec((RB, 1), lambda i: (i, 0)),
            pl.BlockSpec((RB, 1), lambda i: (i, 0)),
        ],
        out_specs=[
            pl.BlockSpec((NC, RB, H), lambda i: (0, i, 0)),
            pl.BlockSpec((RB, 1), lambda i: (i, 0)),
            pl.BlockSpec((RB, 1), lambda i: (i, 0)),
        ],
        out_shape=[
            jax.ShapeDtypeStruct((NC, NP, H), jnp.float32),
            jax.ShapeDtypeStruct((NP, 1), jnp.float32),
            jax.ShapeDtypeStruct((NP, 1), jnp.float32),
        ],
    )(x, cnt_out, cnt_in)


def _layer_body(a_ref, sin_ref, sout_ref, w_ref, b_ref, g_ref, be_ref, h_ref):
    a = jnp.concatenate([a_ref[0], a_ref[1]], axis=1) * sin_ref[...]
    z = jnp.dot(a, w_ref[...], preferred_element_type=jnp.float32) + b_ref[...]
    mu = jnp.mean(z, axis=-1, keepdims=True)
    var = jnp.mean((z - mu) ** 2, axis=-1, keepdims=True)
    f = (z - mu) / jnp.sqrt(var + 1e-5) * g_ref[...] + be_ref[...]
    f = jnp.maximum(f, 0.0) * sout_ref[...]
    h_ref[0] = f[:, :H]
    h_ref[1] = f[:, H:]


def _layer(agg, sin, sout, w, b, g, be):
    return pl.pallas_call(
        _layer_body,
        grid=(NP // RB,),
        in_specs=[
            pl.BlockSpec((NC, RB, H), lambda i: (0, i, 0)),
            pl.BlockSpec((RB, 1), lambda i: (i, 0)),
            pl.BlockSpec((RB, 1), lambda i: (i, 0)),
            pl.BlockSpec((D, D), lambda i: (0, 0)),
            pl.BlockSpec((1, D), lambda i: (0, 0)),
            pl.BlockSpec((1, D), lambda i: (0, 0)),
            pl.BlockSpec((1, D), lambda i: (0, 0)),
        ],
        out_specs=pl.BlockSpec((NC, RB, H), lambda i: (0, i, 0)),
        out_shape=jax.ShapeDtypeStruct((NC, NP, H), jnp.float32),
    )(agg, sin, sout, w, b.reshape(1, D), g.reshape(1, D), be.reshape(1, D))


FB = 256


def _final_body(a_ref, sin_ref, w_ref, b_ref, g_ref, be_ref, wp_ref, bp_ref, y_ref):
    a = jnp.concatenate([a_ref[0], a_ref[1]], axis=1) * sin_ref[...]
    z = jnp.dot(a, w_ref[...], preferred_element_type=jnp.float32) + b_ref[...]
    mu = jnp.mean(z, axis=-1, keepdims=True)
    var = jnp.mean((z - mu) ** 2, axis=-1, keepdims=True)
    f = (z - mu) / jnp.sqrt(var + 1e-5) * g_ref[...] + be_ref[...]
    f = jnp.maximum(f, 0.0)
    y_ref[...] = (
        jnp.dot(f, wp_ref[...], preferred_element_type=jnp.float32) + bp_ref[...]
    )


def _final(sel, sin_sel, w, b, g, be, wp_pad, bp):
    return pl.pallas_call(
        _final_body,
        grid=(NGP // FB,),
        in_specs=[
            pl.BlockSpec((NC, FB, H), lambda i: (0, i, 0)),
            pl.BlockSpec((FB, 1), lambda i: (i, 0)),
            pl.BlockSpec((D, D), lambda i: (0, 0)),
            pl.BlockSpec((1, D), lambda i: (0, 0)),
            pl.BlockSpec((1, D), lambda i: (0, 0)),
            pl.BlockSpec((1, D), lambda i: (0, 0)),
            pl.BlockSpec((D, H), lambda i: (0, 0)),
            pl.BlockSpec((1, 1), lambda i: (0, 0)),
        ],
        out_specs=pl.BlockSpec((FB, H), lambda i: (i, 0)),
        out_shape=jax.ShapeDtypeStruct((NGP, H), jnp.float32),
    )(sel, sin_sel, w, b.reshape(1, D), g.reshape(1, D), be.reshape(1, D),
      wp_pad, bp.reshape(1, 1))


def kernel(features, edge_index, W0, b0, g0, be0, W1, b1, g1, be1,
           W2, b2, g2, be2, Wp, bp):
    src = edge_index[0]
    dst = edge_index[1]
    fill = jnp.full((EPAD,), TRASH, jnp.int32)
    srcp = jnp.concatenate([src, fill]).reshape(NC * NS, NCH, CHUNK)
    dstp = jnp.concatenate([dst, fill]).reshape(NC * NS, NCH, CHUNK)
    ei = jnp.stack([srcp, dstp])

    x = features.reshape(NV, D)
    x = jnp.pad(x, ((0, NP - NV), (0, 0)))

    ei_f = ei.reshape(2, NC * NS, GE)
    counts = _degrees(ei_f)
    cnt_out = counts[0][:, None]
    cnt_in = counts[1][:, None]

    h, sin, sout = _prep(x, cnt_out, cnt_in)

    Ws = [W0, W1, W2]
    bs = [b0, b1, b2]
    gs = [g0, g1, g2]
    bes = [be0, be1, be2]
    for i in range(2):
        agg = _spmm(h, ei_f)
        h = _layer(agg, sin, sout, Ws[i], bs[i], gs[i], bes[i])

    vi = jnp.arange(NP, dtype=jnp.int32)
    tab = jnp.where((vi % NPG == 0) & (vi < NV), vi // NPG, CTRASH)
    tab = tab.astype(jnp.int32)
    sel = _spmm_first(h, ei_f, tab)
    sin_sel = sin[:NV].reshape(NG, NPG)[:, 0:1]
    sin_sel = jnp.pad(sin_sel, ((0, NGP - NG), (0, 0)))

    wp_pad = jnp.pad(Wp, ((0, 0), (0, H - 1)))
    y = _final(sel, sin_sel, W2, b2, g2, be2, wp_pad, bp)
    return y[:NG, 0:1]


# speedup vs baseline: 1.1625x; 1.1625x over previous
import functools

import jax
import jax.numpy as jnp
from jax import lax
from jax.experimental import pallas as pl
from jax.experimental.pallas import tpu as pltpu
from jax.experimental.pallas import tpu_sc as plsc

NG = 1112
NPG = 9
NV = NG * NPG
NE = NV * 16
D = 256
H = 128
NC = 2
NS = 16
NP = 10240
TRASH = NV
CHUNK = 128
NCH = 40
EPAD = NC * NS * NCH * CHUNK - NE
STRIPE = NP // NS


def _mesh():
    return plsc.VectorSubcoreMesh(
        core_axis_name="c", subcore_axis_name="s", num_cores=NC, num_subcores=NS
    )


GE = NCH * CHUNK


def _deg_body(ei_hbm, cnt_hbm, idx_v, hist_v, red_v, out_v, red_sh):
    c = lax.axis_index("c")
    s = lax.axis_index("s")

    def zero(i, carry):
        hist_v[pl.ds(i * 16, 16)] = jnp.zeros((16,), jnp.float32)
        return carry

    lax.fori_loop(0, NP // 16, zero, 0)
    ones = jnp.ones((16,), jnp.float32)
    for gi in range(2):
        g = s + NS * gi
        pltpu.sync_copy(ei_hbm.at[c, g], idx_v)

        def chunk(j, carry):
            idx = idx_v[pl.ds(j * 16, 16)]
            plsc.addupdate_scatter(hist_v, [idx], ones)
            return carry

        lax.fori_loop(0, GE // 16, chunk, 0)
    pltpu.sync_copy(hist_v, red_sh.at[s])
    plsc.subcore_barrier()
    pltpu.sync_copy(red_sh.at[:, pl.ds(s * STRIPE, STRIPE)], red_v)

    def red(k, carry):
        acc = jnp.zeros((16,), jnp.float32)
        for r in range(NS):
            acc = acc + red_v[r, pl.ds(k * 16, 16)]
        out_v[pl.ds(k * 16, 16)] = acc
        return carry

    lax.fori_loop(0, STRIPE // 16, red, 0)
    pltpu.sync_copy(out_v, cnt_hbm.at[c, pl.ds(s * STRIPE, STRIPE)])


def _degrees(ei_flat):
    return pl.kernel(
        _deg_body,
        out_type=jax.ShapeDtypeStruct((NC, NP), jnp.float32),
        mesh=_mesh(),
        scratch_types=[
            pltpu.VMEM((GE,), jnp.int32),
            pltpu.VMEM((NP,), jnp.float32),
            pltpu.VMEM((NS, STRIPE), jnp.float32),
            pltpu.VMEM((STRIPE,), jnp.float32),
            pltpu.VMEM_SHARED((NS, NP), jnp.float32),
        ],
        compiler_params=pltpu.CompilerParams(needs_layout_passes=False),
    )(ei_flat)


NT = 2 * NCH


SCH = 64
NB = 4
NCHS = GE // SCH


def _spmm_body4(h_hbm, ei_hbm, z_hbm, agg_hbm, sidx, didx,
                rows_0, rows_1, rows_2, rows_3, agg_sh,
                gs0, gs1, gs2, gs3, ss0, ss1, ss2, ss3):
    c = lax.axis_index("c")
    s = lax.axis_index("s")
    rows = (rows_0, rows_1, rows_2, rows_3)
    gsems = (gs0, gs1, gs2, gs3)
    ssems = (ss0, ss1, ss2, ss3)
    pltpu.sync_copy(z_hbm, agg_sh.at[pl.ds(s * STRIPE, STRIPE)])
    plsc.subcore_barrier()

    def gather(j, k):
        pltpu.async_copy(h_hbm.at[c].at[sidx.at[pl.ds(j * SCH, SCH)]],
                         rows[k], gsems[k])

    def gwait(j, k):
        pltpu.make_async_copy(h_hbm.at[c].at[sidx.at[pl.ds(j * SCH, SCH)]],
                              rows[k], gsems[k]).wait()

    def sstart(j, k):
        pltpu.async_copy(rows[k], agg_sh.at[didx.at[pl.ds(j * SCH, SCH)]],
                         ssems[k], add=True)

    def swait(j, k):
        pltpu.make_async_copy(rows[k], agg_sh.at[didx.at[pl.ds(j * SCH, SCH)]],
                              ssems[k]).wait()

    for gi in range(2):
        g = s + NS * gi
        pltpu.sync_copy(ei_hbm.at[0, g], sidx)
        pltpu.sync_copy(ei_hbm.at[1, g], didx)
        gather(0, 0)
        gather(1, 1)
        gather(2, 2)

        def body(t, carry):
            for k in range(NB):
                j = NB * t + k
                gwait(j, k)
                sstart(j, k)
                k3 = (k + 3) % NB

                @pl.when(j + 3 < NCHS)
                def _():
                    @pl.when(j >= 1)
                    def _():
                        swait(j - 1, k3)

                    gather(j + 3, k3)

            return carry

        lax.fori_loop(0, NCHS // NB, body, 0)
        for k in range(NB):
            swait(NCHS - NB + k, k)
    plsc.subcore_barrier()
    pltpu.sync_copy(
        agg_sh.at[pl.ds(s * STRIPE, STRIPE)],
        agg_hbm.at[c, pl.ds(s * STRIPE, STRIPE)],
    )


def _spmm(h, ei):
    return pl.kernel(
        _spmm_body4,
        out_type=jax.ShapeDtypeStruct((NC, NP, H), jnp.float32),
        mesh=_mesh(),
        scratch_types=[
            pltpu.VMEM((GE,), jnp.int32),
            pltpu.VMEM((GE,), jnp.int32),
            pltpu.VMEM((SCH, H), jnp.float32),
            pltpu.VMEM((SCH, H), jnp.float32),
            pltpu.VMEM((SCH, H), jnp.float32),
            pltpu.VMEM((SCH, H), jnp.float32),
            pltpu.VMEM_SHARED((NP, H), jnp.float32),
            pltpu.SemaphoreType.DMA,
            pltpu.SemaphoreType.DMA,
            pltpu.SemaphoreType.DMA,
            pltpu.SemaphoreType.DMA,
            pltpu.SemaphoreType.DMA,
            pltpu.SemaphoreType.DMA,
            pltpu.SemaphoreType.DMA,
            pltpu.SemaphoreType.DMA,
        ],
    )(h, ei, jnp.zeros((STRIPE, H), jnp.float32))


NGP = 1280
CTRASH = NGP - 1
GSTRIPE = NGP // NS


def _spmm_first_body(h_hbm, ei_hbm, tab_hbm, z_hbm, agg_hbm, sidx_f, didx_f,
                     csrc, cdst, tab_v, rows_a, rows_b, agg_sh,
                     gsem_a, gsem_b, ssem_a, ssem_b):
    c = lax.axis_index("c")
    s = lax.axis_index("s")
    pltpu.sync_copy(z_hbm.at[pl.ds(0, GSTRIPE)],
                    agg_sh.at[pl.ds(s * GSTRIPE, GSTRIPE)])
    pltpu.sync_copy(tab_hbm, tab_v)

    def fill(i, carry):
        csrc[pl.ds(i * 16, 16)] = jnp.zeros((16,), jnp.int32)
        cdst[pl.ds(i * 16, 16)] = jnp.full((16,), CTRASH, jnp.int32)
        return carry

    lax.fori_loop(0, (2 * GE) // 16, fill, 0)

    def compact(i, o):
        s16 = sidx_f[pl.ds(i * 16, 16)]
        d16 = didx_f[pl.ds(i * 16, 16)]
        dv = plsc.load_gather(tab_v, [d16])
        mask = dv != CTRASH
        plsc.store_compressed(csrc.at[pl.ds(o, 16)], s16, mask=mask)
        plsc.store_compressed(cdst.at[pl.ds(o, 16)], dv, mask=mask)
        return o + jnp.max(plsc.all_reduce_population_count(mask))

    o = jnp.int32(0)
    for gi in range(2):
        g = s + NS * gi
        pltpu.sync_copy(ei_hbm.at[0, g], sidx_f)
        pltpu.sync_copy(ei_hbm.at[1, g], didx_f)
        o = lax.fori_loop(0, GE // 16, compact, o)
    nch = (o + CHUNK - 1) // CHUNK
    plsc.subcore_barrier()

    def chunk(j, carry):
        pltpu.sync_copy(h_hbm.at[c].at[csrc.at[pl.ds(j * CHUNK, CHUNK)]],
                        rows_a)
        pltpu.sync_copy(rows_a, agg_sh.at[cdst.at[pl.ds(j * CHUNK, CHUNK)]],
                        add=True)
        return carry

    lax.fori_loop(0, nch, chunk, 0)
    plsc.subcore_barrier()
    pltpu.sync_copy(
        agg_sh.at[pl.ds(s * GSTRIPE, GSTRIPE)],
        agg_hbm.at[c, pl.ds(s * GSTRIPE, GSTRIPE)],
    )


def _spmm_first(h, ei_flat, tab):
    return pl.kernel(
        _spmm_first_body,
        out_type=jax.ShapeDtypeStruct((NC, NGP, H), jnp.float32),
        mesh=_mesh(),
        scratch_types=[
            pltpu.VMEM((GE,), jnp.int32),
            pltpu.VMEM((GE,), jnp.int32),
            pltpu.VMEM((2 * GE,), jnp.int32),
            pltpu.VMEM((2 * GE,), jnp.int32),
            pltpu.VMEM((NP,), jnp.int32),
            pltpu.VMEM((CHUNK, H), jnp.float32),
            pltpu.VMEM((CHUNK, H), jnp.float32),
            pltpu.VMEM_SHARED((NGP, H), jnp.float32),
            pltpu.SemaphoreType.DMA,
            pltpu.SemaphoreType.DMA,
            pltpu.SemaphoreType.DMA,
            pltpu.SemaphoreType.DMA,
        ],
        compiler_params=pltpu.CompilerParams(needs_layout_passes=False),
    )(h, ei_flat, tab, jnp.zeros((GSTRIPE, H), jnp.float32))


RB = 512


def _prep_body(x_ref, co_ref, ci_ref, h_ref, sin_ref, sout_ref):
    so = lax.rsqrt(jnp.maximum(co_ref[...], 1.0))
    si = lax.rsqrt(jnp.maximum(ci_ref[...], 1.0))
    sin_ref[...] = si
    sout_ref[...] = so
    xs = x_ref[...] * so
    h_ref[0] = xs[:, :H]
    h_ref[1] = xs[:, H:]


def _prep(x, cnt_out, cnt_in):
    return pl.pallas_call(
        _prep_body,
        grid=(NP // RB,),
        in_specs=[
            pl.BlockSpec((RB, D), lambda i: (i, 0)),
            pl.BlockSpec((RB, 1), lambda i: (i, 0)),
            pl.BlockSpec((RB, 1), lambda i: (i, 0)),
        ],
        out_specs=[
            pl.BlockSpec((NC, RB, H), lambda i: (0, i, 0)),
            pl.BlockSpec((RB, 1), lambda i: (i, 0)),
            pl.BlockSpec((RB, 1), lambda i: (i, 0)),
        ],
        out_shape=[
            jax.ShapeDtypeStruct((NC, NP, H), jnp.float32),
            jax.ShapeDtypeStruct((NP, 1), jnp.float32),
            jax.ShapeDtypeStruct((NP, 1), jnp.float32),
        ],
    )(x, cnt_out, cnt_in)


def _layer_body(a_ref, sin_ref, sout_ref, w_ref, b_ref, g_ref, be_ref, h_ref):
    a = jnp.concatenate([a_ref[0], a_ref[1]], axis=1) * sin_ref[...]
    z = jnp.dot(a, w_ref[...], preferred_element_type=jnp.float32) + b_ref[...]
    mu = jnp.mean(z, axis=-1, keepdims=True)
    var = jnp.mean((z - mu) ** 2, axis=-1, keepdims=True)
    f = (z - mu) / jnp.sqrt(var + 1e-5) * g_ref[...] + be_ref[...]
    f = jnp.maximum(f, 0.0) * sout_ref[...]
    h_ref[0] = f[:, :H]
    h_ref[1] = f[:, H:]


def _layer(agg, sin, sout, w, b, g, be):
    return pl.pallas_call(
        _layer_body,
        grid=(NP // RB,),
        in_specs=[
            pl.BlockSpec((NC, RB, H), lambda i: (0, i, 0)),
            pl.BlockSpec((RB, 1), lambda i: (i, 0)),
            pl.BlockSpec((RB, 1), lambda i: (i, 0)),
            pl.BlockSpec((D, D), lambda i: (0, 0)),
            pl.BlockSpec((1, D), lambda i: (0, 0)),
            pl.BlockSpec((1, D), lambda i: (0, 0)),
            pl.BlockSpec((1, D), lambda i: (0, 0)),
        ],
        out_specs=pl.BlockSpec((NC, RB, H), lambda i: (0, i, 0)),
        out_shape=jax.ShapeDtypeStruct((NC, NP, H), jnp.float32),
    )(agg, sin, sout, w, b.reshape(1, D), g.reshape(1, D), be.reshape(1, D))


FB = 256


def _final_body(a_ref, sin_ref, w_ref, b_ref, g_ref, be_ref, wp_ref, bp_ref, y_ref):
    a = jnp.concatenate([a_ref[0], a_ref[1]], axis=1) * sin_ref[...]
    z = jnp.dot(a, w_ref[...], preferred_element_type=jnp.float32) + b_ref[...]
    mu = jnp.mean(z, axis=-1, keepdims=True)
    var = jnp.mean((z - mu) ** 2, axis=-1, keepdims=True)
    f = (z - mu) / jnp.sqrt(var + 1e-5) * g_ref[...] + be_ref[...]
    f = jnp.maximum(f, 0.0)
    y_ref[...] = (
        jnp.dot(f, wp_ref[...], preferred_element_type=jnp.float32) + bp_ref[...]
    )


def _final(sel, sin_sel, w, b, g, be, wp_pad, bp):
    return pl.pallas_call(
        _final_body,
        grid=(NGP // FB,),
        in_specs=[
            pl.BlockSpec((NC, FB, H), lambda i: (0, i, 0)),
            pl.BlockSpec((FB, 1), lambda i: (i, 0)),
            pl.BlockSpec((D, D), lambda i: (0, 0)),
            pl.BlockSpec((1, D), lambda i: (0, 0)),
            pl.BlockSpec((1, D), lambda i: (0, 0)),
            pl.BlockSpec((1, D), lambda i: (0, 0)),
            pl.BlockSpec((D, H), lambda i: (0, 0)),
            pl.BlockSpec((1, 1), lambda i: (0, 0)),
        ],
        out_specs=pl.BlockSpec((FB, H), lambda i: (i, 0)),
        out_shape=jax.ShapeDtypeStruct((NGP, H), jnp.float32),
    )(sel, sin_sel, w, b.reshape(1, D), g.reshape(1, D), be.reshape(1, D),
      wp_pad, bp.reshape(1, 1))


def kernel(features, edge_index, W0, b0, g0, be0, W1, b1, g1, be1,
           W2, b2, g2, be2, Wp, bp):
    src = edge_index[0]
    dst = edge_index[1]
    fill = jnp.full((EPAD,), TRASH, jnp.int32)
    srcp = jnp.concatenate([src, fill]).reshape(NC * NS, NCH, CHUNK)
    dstp = jnp.concatenate([dst, fill]).reshape(NC * NS, NCH, CHUNK)
    ei = jnp.stack([srcp, dstp])

    x = features.reshape(NV, D)
    x = jnp.pad(x, ((0, NP - NV), (0, 0)))

    ei_f = ei.reshape(2, NC * NS, GE)
    counts = _degrees(ei_f)
    cnt_out = counts[0][:, None]
    cnt_in = counts[1][:, None]

    h, sin, sout = _prep(x, cnt_out, cnt_in)

    Ws = [W0, W1, W2]
    bs = [b0, b1, b2]
    gs = [g0, g1, g2]
    bes = [be0, be1, be2]
    for i in range(2):
        agg = _spmm(h, ei_f)
        h = _layer(agg, sin, sout, Ws[i], bs[i], gs[i], bes[i])

    vi = jnp.arange(NP, dtype=jnp.int32)
    tab = jnp.where((vi % NPG == 0) & (vi < NV), vi // NPG, CTRASH)
    tab = tab.astype(jnp.int32)
    sel = _spmm_first(h, ei_f, tab)
    sin_sel = sin[:NV].reshape(NG, NPG)[:, 0:1]
    sin_sel = jnp.pad(sin_sel, ((0, NGP - NG), (0, 0)))

    wp_pad = jnp.pad(Wp, ((0, 0), (0, H - 1)))
    y = _final(sel, sin_sel, W2, b2, g2, be2, wp_pad, bp)
    return y[:NG, 0:1]
